# baseline (speedup 1.0000x reference)
import functools
import math

import jax
import jax.numpy as jnp
from jax import lax
from jax.experimental import pallas as pl
from jax.experimental.pallas import tpu as pltpu
from jax.experimental.pallas import tpu_sc as plsc

VOCAB = 1_000_000
D_MODEL = 64
SCALE = math.sqrt(D_MODEL)

NUM_CORES = 2
NUM_SUBCORES = 16
LANES = 16
NUM_WORKERS = NUM_CORES * NUM_SUBCORES

CHUNK = 128
NBUF = 4


def _make_lookup(batch_flat: int):
  assert batch_flat % (NUM_WORKERS * CHUNK * NBUF) == 0
  per_worker = batch_flat // NUM_WORKERS
  n_chunks = per_worker // CHUNK

  mesh = plsc.VectorSubcoreMesh(
      core_axis_name="c", subcore_axis_name="s",
      num_cores=NUM_CORES, num_subcores=NUM_SUBCORES)

  @functools.partial(
      pl.kernel,
      out_type=jax.ShapeDtypeStruct((batch_flat, 128), jnp.float32),
      mesh=mesh,
      scratch_types=[
          pltpu.VMEM((n_chunks, CHUNK), jnp.int32),
          pltpu.VMEM((NBUF, CHUNK, 128), jnp.float32),
          pltpu.SemaphoreType.DMA((NBUF,)),
          pltpu.SemaphoreType.DMA((NBUF,)),
      ],
      compiler_params=pltpu.CompilerParams(use_tc_tiling_on_sc=True,
                                           needs_layout_passes=False),
  )
  def lookup(table_hbm, idx_hbm, out_hbm, idx_all, rows, gsem, ssem):
    wid = lax.axis_index("s") * NUM_CORES + lax.axis_index("c")
    base = wid * per_worker
    pltpu.sync_copy(idx_hbm.at[pl.ds(wid * n_chunks, n_chunks)], idx_all)

    @pl.loop(0, n_chunks, step=NBUF)
    def _group(g0):
      gathers = [
          pltpu.async_copy(
              table_hbm.at[idx_all.at[g0 + b]], rows.at[b], gsem.at[b])
          for b in range(NBUF)
      ]
      stores = []
      for b in range(NBUF):
        gathers[b].wait()
        row_buf = rows.at[b]

        @plsc.parallel_loop(0, CHUNK, unroll=4)
        def _scale(r, row_buf=row_buf):
          for j in range(D_MODEL // LANES):
            sl = (r, pl.ds(j * LANES, LANES))
            row_buf[sl] = row_buf[sl] * SCALE

        stores.append(
            pltpu.async_copy(
                row_buf, out_hbm.at[pl.ds(base + (g0 + b) * CHUNK, CHUNK)],
                ssem.at[b]))
      for st in stores:
        st.wait()

  return lookup


def kernel(x, table):
  batch_shape = x.shape
  x_flat = x.reshape(-1).astype(jnp.int32)
  idx2d = x_flat.reshape(-1, CHUNK)
  eye_pad = jnp.concatenate(
      [jnp.eye(D_MODEL, dtype=jnp.float32),
       jnp.zeros((D_MODEL, 128 - D_MODEL), jnp.float32)], axis=1)
  twide = jax.lax.dot(table, eye_pad, precision=jax.lax.Precision.HIGHEST)
  out = _make_lookup(x_flat.shape[0])(twide, idx2d)
  out3 = out.reshape(*batch_shape, 128)
  return out3[..., :D_MODEL]
